# baseline (speedup 1.0000x reference)
import jax
import jax.numpy as jnp
from jax.experimental import pallas as pl
from jax.experimental.pallas import tpu as pltpu

TOPN = 5
DEPTH = 4
WPAD = 64


def _topk_kernel(s_ref, idx_ref):
    x = s_ref[:, :]
    S = x.shape[1]
    iota = jax.lax.broadcasted_iota(jnp.int32, x.shape, 1)
    cols = []
    for _ in range(TOPN):
        m = jnp.max(x, axis=1, keepdims=True)
        am = jnp.min(jnp.where(x >= m, iota, S), axis=1, keepdims=True)
        cols.append(am)
        x = jnp.where(iota == am, -jnp.inf, x)
    idx_ref[:, :] = jnp.concatenate(cols, axis=1)


def _cosine_kernel(idx_ref, qa_ref, story_ref, out_ref, buf_ref, sem):
    B = pl.num_programs(0)
    b = pl.program_id(0)
    Q = qa_ref.shape[1]
    Wq = qa_ref.shape[2]
    W = story_ref.shape[1]
    QN = Q * TOPN

    def copies(step, slot):
        cps = []
        for j in range(QN):
            s = idx_ref[step, j // TOPN, j % TOPN]
            cps.append(pltpu.make_async_copy(
                story_ref.at[s, :, step, :],
                buf_ref.at[slot, j, pl.ds(0, W), :],
                sem.at[slot, j]))
        return cps

    def issue(step):
        @pl.when(step < B)
        def _():
            for c in copies(step, jax.lax.rem(step, DEPTH)):
                c.start()

    @pl.when(b == 0)
    def _():
        for p in range(DEPTH):
            issue(p)

    slot = jax.lax.rem(b, DEPTH)
    for c in copies(b, slot):
        c.wait()

    ones8 = jnp.ones((8, 128), jnp.float32)
    lane = jax.lax.broadcasted_iota(jnp.int32, (1, TOPN * WPAD), 1)
    word_mask = jax.lax.rem(lane, WPAD) < W

    for q in range(Q):
        qa = qa_ref[0, q]
        qa_n = qa * jax.lax.rsqrt(
            jnp.sum(qa * qa, axis=1, keepdims=True) + 1e-6)
        t = buf_ref[slot, q * TOPN:(q + 1) * TOPN]
        t = t.reshape(TOPN * WPAD, t.shape[2])
        dot = jax.lax.dot_general(qa_n, t, (((1,), (1,)), ((), ())),
                                  preferred_element_type=jnp.float32)
        nrm = jax.lax.dot_general(ones8, t * t, (((1,), (1,)), ((), ())),
                                  preferred_element_type=jnp.float32)
        cos = dot * jax.lax.rsqrt(nrm[0:1, :] + 1e-6)
        masked = jnp.where(word_mask, cos, -3e38)
        cmax = jnp.max(masked, axis=1, keepdims=True)
        out_ref[0, q] = jnp.pad(cmax, ((0, 2), (0, 0)),
                                constant_values=-3e38)

    @pl.when(b > 0)
    def _():
        issue(b + DEPTH - 1)


def _epilogue_kernel(alpha_ref, cm_ref, w_ref, out_ref):
    Wq = w_ref.shape[1]
    cm = cm_ref[:, :Wq]
    w = w_ref[:, :]
    wn = w / (jnp.sum(w, axis=1, keepdims=True) + 1e-6)
    out_ref[:, :] = jnp.sum(cm * wn, axis=1, keepdims=True) * alpha_ref[0]


def kernel(sentence_scores, story_word_embedding, qa_embedding, qa_weights,
           alpha, beta):
    B, S, Q = sentence_scores.shape
    W, D = story_word_embedding.shape[2], story_word_embedding.shape[3]
    Wq = qa_embedding.shape[2]

    scores = jnp.transpose(sentence_scores, (0, 2, 1)).reshape(B * Q, S)
    idx = pl.pallas_call(
        _topk_kernel,
        out_shape=jax.ShapeDtypeStruct((B * Q, TOPN), jnp.int32),
    )(scores)
    idx = idx.reshape(B, Q, TOPN)

    cm = pl.pallas_call(
        _cosine_kernel,
        grid_spec=pltpu.PrefetchScalarGridSpec(
            num_scalar_prefetch=1,
            grid=(B,),
            in_specs=[
                pl.BlockSpec((1, Q, Wq, D), lambda b, i_r: (b, 0, 0, 0)),
                pl.BlockSpec(memory_space=pltpu.HBM),
            ],
            out_specs=pl.BlockSpec((1, Q, Wq + 2, 1), lambda b, i_r: (b, 0, 0, 0)),
            scratch_shapes=[
                pltpu.VMEM((DEPTH, Q * TOPN, WPAD, D), jnp.float32),
                pltpu.SemaphoreType.DMA((DEPTH, Q * TOPN)),
            ],
        ),
        out_shape=jax.ShapeDtypeStruct((B, Q, Wq + 2, 1), jnp.float32),
    )(idx, qa_embedding, jnp.transpose(story_word_embedding, (1, 2, 0, 3)))

    cm2 = cm.reshape(B * Q, Wq + 2)
    qa_w = qa_weights.reshape(B * Q, Wq)
    alpha_arr = jnp.reshape(alpha, (1,)).astype(jnp.float32)
    out = pl.pallas_call(
        _epilogue_kernel,
        grid_spec=pltpu.PrefetchScalarGridSpec(
            num_scalar_prefetch=1,
            grid=(1,),
            in_specs=[
                pl.BlockSpec((B * Q, Wq + 2), lambda i, a_r: (0, 0)),
                pl.BlockSpec((B * Q, Wq), lambda i, a_r: (0, 0)),
            ],
            out_specs=pl.BlockSpec((B * Q, 1), lambda i, a_r: (0, 0)),
        ),
        out_shape=jax.ShapeDtypeStruct((B * Q, 1), jnp.float32),
    )(alpha_arr, cm2, qa_w)
    return out.reshape(B, Q) + 0.0 * beta


# speedup vs baseline: 1.0351x; 1.0351x over previous
import jax
import jax.numpy as jnp
from jax.experimental import pallas as pl
from jax.experimental.pallas import tpu as pltpu

TOPN = 5
DEPTH = 8
WPAD = 64


def _topk_kernel(s_ref, idx_ref):
    x = s_ref[:, :]
    S = x.shape[1]
    iota = jax.lax.broadcasted_iota(jnp.int32, x.shape, 1)
    cols = []
    for _ in range(TOPN):
        m = jnp.max(x, axis=1, keepdims=True)
        am = jnp.min(jnp.where(x >= m, iota, S), axis=1, keepdims=True)
        cols.append(am)
        x = jnp.where(iota == am, -jnp.inf, x)
    idx_ref[:, :] = jnp.concatenate(cols, axis=1)


def _cosine_kernel(idx_ref, qa_ref, story_ref, out_ref, buf_ref, sem):
    B = pl.num_programs(0)
    b = pl.program_id(0)
    Q = qa_ref.shape[1]
    Wq = qa_ref.shape[2]
    W = story_ref.shape[1]
    QN = Q * TOPN

    def copies(step, slot):
        cps = []
        for j in range(QN):
            s = idx_ref[step, j // TOPN, j % TOPN]
            cps.append(pltpu.make_async_copy(
                story_ref.at[s, :, step, :],
                buf_ref.at[slot, j, pl.ds(0, W), :],
                sem.at[slot, j]))
        return cps

    def issue(step):
        @pl.when(step < B)
        def _():
            for c in copies(step, jax.lax.rem(step, DEPTH)):
                c.start()

    @pl.when(b == 0)
    def _():
        for p in range(DEPTH):
            issue(p)

    @pl.when(b > 0)
    def _():
        issue(b + DEPTH - 1)

    slot = jax.lax.rem(b, DEPTH)
    for c in copies(b, slot):
        c.wait()

    ones8 = jnp.ones((8, 128), jnp.float32)
    lane = jax.lax.broadcasted_iota(jnp.int32, (1, TOPN * WPAD), 1)
    word_mask = jax.lax.rem(lane, WPAD) < W

    for q in range(Q):
        qa = qa_ref[0, q]
        qa_n = qa * jax.lax.rsqrt(
            jnp.sum(qa * qa, axis=1, keepdims=True) + 1e-6)
        t = buf_ref[slot, q * TOPN:(q + 1) * TOPN]
        t = t.reshape(TOPN * WPAD, t.shape[2])
        dot = jax.lax.dot_general(qa_n, t, (((1,), (1,)), ((), ())),
                                  preferred_element_type=jnp.float32)
        nrm = jax.lax.dot_general(ones8, t * t, (((1,), (1,)), ((), ())),
                                  preferred_element_type=jnp.float32)
        cos = dot * jax.lax.rsqrt(nrm[0:1, :] + 1e-6)
        masked = jnp.where(word_mask, cos, -3e38)
        cmax = jnp.max(masked, axis=1, keepdims=True)
        out_ref[0, q] = jnp.pad(cmax, ((0, 2), (0, 0)),
                                constant_values=-3e38)


def _epilogue_kernel(alpha_ref, cm_ref, w_ref, out_ref):
    Wq = w_ref.shape[1]
    cm = cm_ref[:, :Wq]
    w = w_ref[:, :]
    wn = w / (jnp.sum(w, axis=1, keepdims=True) + 1e-6)
    out_ref[:, :] = jnp.sum(cm * wn, axis=1, keepdims=True) * alpha_ref[0]


def kernel(sentence_scores, story_word_embedding, qa_embedding, qa_weights,
           alpha, beta):
    B, S, Q = sentence_scores.shape
    W, D = story_word_embedding.shape[2], story_word_embedding.shape[3]
    Wq = qa_embedding.shape[2]

    scores = jnp.transpose(sentence_scores, (0, 2, 1)).reshape(B * Q, S)
    idx = pl.pallas_call(
        _topk_kernel,
        out_shape=jax.ShapeDtypeStruct((B * Q, TOPN), jnp.int32),
    )(scores)
    idx = idx.reshape(B, Q, TOPN)

    cm = pl.pallas_call(
        _cosine_kernel,
        grid_spec=pltpu.PrefetchScalarGridSpec(
            num_scalar_prefetch=1,
            grid=(B,),
            in_specs=[
                pl.BlockSpec((1, Q, Wq, D), lambda b, i_r: (b, 0, 0, 0)),
                pl.BlockSpec(memory_space=pltpu.HBM),
            ],
            out_specs=pl.BlockSpec((1, Q, Wq + 2, 1), lambda b, i_r: (b, 0, 0, 0)),
            scratch_shapes=[
                pltpu.VMEM((DEPTH, Q * TOPN, WPAD, D), jnp.float32),
                pltpu.SemaphoreType.DMA((DEPTH, Q * TOPN)),
            ],
        ),
        out_shape=jax.ShapeDtypeStruct((B, Q, Wq + 2, 1), jnp.float32),
    )(idx, qa_embedding, jnp.transpose(story_word_embedding, (1, 2, 0, 3)))

    cm2 = cm.reshape(B * Q, Wq + 2)
    qa_w = qa_weights.reshape(B * Q, Wq)
    alpha_arr = jnp.reshape(alpha, (1,)).astype(jnp.float32)
    out = pl.pallas_call(
        _epilogue_kernel,
        grid_spec=pltpu.PrefetchScalarGridSpec(
            num_scalar_prefetch=1,
            grid=(1,),
            in_specs=[
                pl.BlockSpec((B * Q, Wq + 2), lambda i, a_r: (0, 0)),
                pl.BlockSpec((B * Q, Wq), lambda i, a_r: (0, 0)),
            ],
            out_specs=pl.BlockSpec((B * Q, 1), lambda i, a_r: (0, 0)),
        ),
        out_shape=jax.ShapeDtypeStruct((B * Q, 1), jnp.float32),
    )(alpha_arr, cm2, qa_w)
    return out.reshape(B, Q) + 0.0 * beta


# speedup vs baseline: 1.3600x; 1.3138x over previous
import jax
import jax.numpy as jnp
from jax.experimental import pallas as pl
from jax.experimental.pallas import tpu as pltpu

TOPN = 5
DEPTH = 4
WPAD = 64
BPS = 4


def _topk_kernel(s_ref, idx_ref):
    x = s_ref[:, :]
    S = x.shape[1]
    iota = jax.lax.broadcasted_iota(jnp.int32, x.shape, 1)
    cols = []
    for _ in range(TOPN):
        m = jnp.max(x, axis=1, keepdims=True)
        am = jnp.min(jnp.where(x >= m, iota, S), axis=1, keepdims=True)
        cols.append(am)
        x = jnp.where(iota == am, -jnp.inf, x)
    idx_ref[:, :] = jnp.concatenate(cols, axis=1)


def _cosine_kernel(idx_ref, qa_ref, story_ref, out_ref, buf_ref, sem):
    nsteps = pl.num_programs(0)
    step0 = pl.program_id(0)
    Q = qa_ref.shape[1]
    Wq = qa_ref.shape[2]
    W = story_ref.shape[1]
    QN = Q * TOPN

    def copies(step, slot):
        cps = []
        for i in range(BPS):
            for j in range(QN):
                s = idx_ref[step * BPS + i, j // TOPN, j % TOPN]
                cps.append(pltpu.make_async_copy(
                    story_ref.at[s, :, step * BPS + i, :],
                    buf_ref.at[slot, i, j, pl.ds(0, W), :],
                    sem.at[slot, i, j]))
        return cps

    def issue(step):
        @pl.when(step < nsteps)
        def _():
            for c in copies(step, jax.lax.rem(step, DEPTH)):
                c.start()

    @pl.when(step0 == 0)
    def _():
        for p in range(DEPTH):
            issue(p)

    @pl.when(step0 > 0)
    def _():
        issue(step0 + DEPTH - 1)

    slot = jax.lax.rem(step0, DEPTH)
    for c in copies(step0, slot):
        c.wait()

    ones8 = jnp.ones((8, 128), jnp.float32)
    lane = jax.lax.broadcasted_iota(jnp.int32, (1, TOPN * WPAD), 1)
    word_mask = jax.lax.rem(lane, WPAD) < W

    for i in range(BPS):
        for q in range(Q):
            qa = qa_ref[i, q]
            qa_n = qa * jax.lax.rsqrt(
                jnp.sum(qa * qa, axis=1, keepdims=True) + 1e-6)
            t = buf_ref[slot, i, q * TOPN:(q + 1) * TOPN]
            t = t.reshape(TOPN * WPAD, t.shape[2])
            dot = jax.lax.dot_general(qa_n, t, (((1,), (1,)), ((), ())),
                                      preferred_element_type=jnp.float32)
            nrm = jax.lax.dot_general(ones8, t * t, (((1,), (1,)), ((), ())),
                                      preferred_element_type=jnp.float32)
            cos = dot * jax.lax.rsqrt(nrm[0:1, :] + 1e-6)
            masked = jnp.where(word_mask, cos, -3e38)
            cmax = jnp.max(masked, axis=1, keepdims=True)
            out_ref[i, q] = jnp.pad(cmax, ((0, 2), (0, 0)),
                                    constant_values=-3e38)


def _epilogue_kernel(alpha_ref, cm_ref, w_ref, out_ref):
    Wq = w_ref.shape[1]
    cm = cm_ref[:, :Wq]
    w = w_ref[:, :]
    wn = w / (jnp.sum(w, axis=1, keepdims=True) + 1e-6)
    out_ref[:, :] = jnp.sum(cm * wn, axis=1, keepdims=True) * alpha_ref[0]


def kernel(sentence_scores, story_word_embedding, qa_embedding, qa_weights,
           alpha, beta):
    B, S, Q = sentence_scores.shape
    W, D = story_word_embedding.shape[2], story_word_embedding.shape[3]
    Wq = qa_embedding.shape[2]

    scores = jnp.transpose(sentence_scores, (0, 2, 1)).reshape(B * Q, S)
    idx = pl.pallas_call(
        _topk_kernel,
        out_shape=jax.ShapeDtypeStruct((B * Q, TOPN), jnp.int32),
    )(scores)
    idx = idx.reshape(B, Q, TOPN)

    cm = pl.pallas_call(
        _cosine_kernel,
        grid_spec=pltpu.PrefetchScalarGridSpec(
            num_scalar_prefetch=1,
            grid=(B // BPS,),
            in_specs=[
                pl.BlockSpec((BPS, Q, Wq, D), lambda b, i_r: (b, 0, 0, 0)),
                pl.BlockSpec(memory_space=pltpu.HBM),
            ],
            out_specs=pl.BlockSpec((BPS, Q, Wq + 2, 1),
                                   lambda b, i_r: (b, 0, 0, 0)),
            scratch_shapes=[
                pltpu.VMEM((DEPTH, BPS, Q * TOPN, WPAD, D), jnp.float32),
                pltpu.SemaphoreType.DMA((DEPTH, BPS, Q * TOPN)),
            ],
        ),
        out_shape=jax.ShapeDtypeStruct((B, Q, Wq + 2, 1), jnp.float32),
    )(idx, qa_embedding, jnp.transpose(story_word_embedding, (1, 2, 0, 3)))

    cm2 = cm.reshape(B * Q, Wq + 2)
    qa_w = qa_weights.reshape(B * Q, Wq)
    alpha_arr = jnp.reshape(alpha, (1,)).astype(jnp.float32)
    out = pl.pallas_call(
        _epilogue_kernel,
        grid_spec=pltpu.PrefetchScalarGridSpec(
            num_scalar_prefetch=1,
            grid=(1,),
            in_specs=[
                pl.BlockSpec((B * Q, Wq + 2), lambda i, a_r: (0, 0)),
                pl.BlockSpec((B * Q, Wq), lambda i, a_r: (0, 0)),
            ],
            out_specs=pl.BlockSpec((B * Q, 1), lambda i, a_r: (0, 0)),
        ),
        out_shape=jax.ShapeDtypeStruct((B * Q, 1), jnp.float32),
    )(alpha_arr, cm2, qa_w)
    return out.reshape(B, Q) + 0.0 * beta
